# baseline (device time: 116032 ns/iter reference)
import jax
import jax.numpy as jnp
from jax import lax
from jax.experimental import pallas as pl
from jax.experimental.pallas import tpu as pltpu

N_DEV = 8
ROWS = 2048
V_PER = 8192
D = 1024
HALF = D // 2
CHUNK = ROWS // N_DEV


def kernel(table, idx):
    def body(table_ref, idx_ref, out_ref, land_r, land_l,
             gather_sems, rs_send, rs_recv, ag_send, ag_recv):
        my = lax.axis_index("i")
        left = lax.rem(my + (N_DEV - 1), N_DEV)
        right = lax.rem(my + 1, N_DEV)

        out_ref[...] = jnp.zeros((ROWS, D), jnp.float32)

        def issue_window(c, slot):
            base = c * CHUNK
            lo = my * V_PER

            def issue_one(j, cnt):
                g = idx_ref[j]
                owned = (g >> 13) == my

                @pl.when(owned)
                def _():
                    pltpu.make_async_copy(
                        table_ref.at[pl.ds(g - lo, 1), :],
                        out_ref.at[pl.ds(j, 1), :],
                        gather_sems.at[slot],
                    ).start()

                return cnt + owned.astype(jnp.int32)

            def gbody(i, cnt):
                j = base + i * 2
                return issue_one(j + 1, issue_one(j, cnt))

            return lax.fori_loop(0, CHUNK // 2, gbody, jnp.int32(0))

        def wait_window(cnt, slot):
            wait_cp = pltpu.make_async_copy(
                table_ref.at[pl.ds(0, 1), :],
                out_ref.at[pl.ds(0, 1), :],
                gather_sems.at[slot],
            )

            def wbody(i, acc):
                wait_cp.wait()
                return acc

            lax.fori_loop(0, cnt, wbody, jnp.int32(0))

        cnt0 = issue_window(my, 0)

        barrier = pltpu.get_barrier_semaphore()
        for nbr in (left, right):
            pl.semaphore_signal(
                barrier, inc=1,
                device_id=(nbr,), device_id_type=pl.DeviceIdType.MESH,
            )
        pl.semaphore_wait(barrier, 2)

        wait_window(cnt0, 0)

        M = 4
        SEG = CHUNK // M
        rs_desc = {}

        def rs_rdma(dirn, s, h):
            if dirn == 0:
                c = lax.rem(my + (N_DEV - s), N_DEV)
                cols = pl.ds(0, HALF)
                land = land_r
                dev = right
            else:
                c = lax.rem(my + s, N_DEV)
                cols = pl.ds(HALF, HALF)
                land = land_l
                dev = left
            return pltpu.make_async_remote_copy(
                src_ref=out_ref.at[pl.ds(c * CHUNK + h * SEG, SEG), cols],
                dst_ref=land.at[s, pl.ds(h * SEG, SEG), :],
                send_sem=rs_send.at[dirn, s, h],
                recv_sem=rs_recv.at[dirn, s, h],
                device_id=(dev,),
                device_id_type=pl.DeviceIdType.MESH,
            )

        for h in range(M):
            for dirn in (0, 1):
                rs_desc[(dirn, 0, h)] = rs_rdma(dirn, 0, h)
                rs_desc[(dirn, 0, h)].start()

        wcnt = {1: issue_window(lax.rem(my + (N_DEV - 1), N_DEV), 1)
                + issue_window(lax.rem(my + 1, N_DEV), 1)}

        for s in range(N_DEV - 1):
            crr = lax.rem(my + (N_DEV - s - 1), N_DEV)
            clr = lax.rem(my + s + 1, N_DEV)
            for h in range(M):
                for dirn in (0, 1):
                    rs_desc[(dirn, s, h)].wait_recv()
                    if dirn == 0 and h == 0 and s + 1 <= 4:
                        wait_window(wcnt[s + 1], s + 1)
                    if dirn == 0:
                        sl = pl.ds(crr * CHUNK + h * SEG, SEG)
                        out_ref[sl, pl.ds(0, HALF)] = (
                            out_ref[sl, pl.ds(0, HALF)]
                            + land_r[s, h * SEG:(h + 1) * SEG]
                        )
                    else:
                        sl = pl.ds(clr * CHUNK + h * SEG, SEG)
                        out_ref[sl, pl.ds(HALF, HALF)] = (
                            out_ref[sl, pl.ds(HALF, HALF)]
                            + land_l[s, h * SEG:(h + 1) * SEG]
                        )
                    if s < N_DEV - 2:
                        rs_desc[(dirn, s + 1, h)] = rs_rdma(dirn, s + 1, h)
                        rs_desc[(dirn, s + 1, h)].start()
            if s < N_DEV - 2 and s + 2 <= 4:
                cnt = issue_window(lax.rem(my + (N_DEV - s - 2), N_DEV), s + 2)
                if s + 2 < 4:
                    cnt = cnt + issue_window(lax.rem(my + s + 2, N_DEV), s + 2)
                wcnt[s + 2] = cnt

        for s in range(N_DEV - 1):
            for h in range(M):
                for dirn in (0, 1):
                    rs_desc[(dirn, s, h)].wait_send()

        ag_desc = {}

        def ag_rdma(dirn, t, h):
            if dirn == 0:
                c = lax.rem(my + (N_DEV + 1 - t), N_DEV)
                cols = pl.ds(0, HALF)
                dev = right
            else:
                c = lax.rem(my + (N_DEV - 1 + t), N_DEV)
                cols = pl.ds(HALF, HALF)
                dev = left
            sl = pl.ds(c * CHUNK + h * SEG, SEG)
            return pltpu.make_async_remote_copy(
                src_ref=out_ref.at[sl, cols],
                dst_ref=out_ref.at[sl, cols],
                send_sem=ag_send.at[dirn, t, h],
                recv_sem=ag_recv.at[dirn, t, h],
                device_id=(dev,),
                device_id_type=pl.DeviceIdType.MESH,
            )

        for dirn in (0, 1):
            for h in range(M):
                ag_desc[(dirn, 0, h)] = ag_rdma(dirn, 0, h)
                ag_desc[(dirn, 0, h)].start()
        for t in range(1, N_DEV - 1):
            for h in range(M):
                for dirn in (0, 1):
                    ag_desc[(dirn, t - 1, h)].wait_recv()
                    ag_desc[(dirn, t, h)] = ag_rdma(dirn, t, h)
                    ag_desc[(dirn, t, h)].start()
        for dirn in (0, 1):
            for h in range(M):
                ag_desc[(dirn, N_DEV - 2, h)].wait_recv()
        for t in range(N_DEV - 1):
            for h in range(M):
                for dirn in (0, 1):
                    ag_desc[(dirn, t, h)].wait_send()

    return pl.pallas_call(
        body,
        out_shape=jax.ShapeDtypeStruct((ROWS, D), jnp.float32),
        in_specs=[
            pl.BlockSpec(memory_space=pl.ANY),
            pl.BlockSpec(memory_space=pltpu.SMEM),
        ],
        out_specs=pl.BlockSpec(memory_space=pltpu.VMEM),
        scratch_shapes=[
            pltpu.VMEM((N_DEV - 1, CHUNK, HALF), jnp.float32),
            pltpu.VMEM((N_DEV - 1, CHUNK, HALF), jnp.float32),
            pltpu.SemaphoreType.DMA((5,)),
            pltpu.SemaphoreType.DMA((2, N_DEV - 1, 4)),
            pltpu.SemaphoreType.DMA((2, N_DEV - 1, 4)),
            pltpu.SemaphoreType.DMA((2, N_DEV - 1, 4)),
            pltpu.SemaphoreType.DMA((2, N_DEV - 1, 4)),
        ],
        compiler_params=pltpu.CompilerParams(collective_id=0),
    )(table, idx)


# device time: 109969 ns/iter; 1.0551x vs baseline; 1.0551x over previous
import jax
import jax.numpy as jnp
from jax import lax
from jax.experimental import pallas as pl
from jax.experimental.pallas import tpu as pltpu

N_DEV = 8
ROWS = 2048
V_PER = 8192
D = 1024
HALF = D // 2
CHUNK = ROWS // N_DEV


def kernel(table, idx):
    def body(table_ref, idx_ref, out_ref, land_r, land_l,
             gather_sems, rs_send, rs_recv, ag_send, ag_recv):
        my = lax.axis_index("i")
        left = lax.rem(my + (N_DEV - 1), N_DEV)
        right = lax.rem(my + 1, N_DEV)

        barrier = pltpu.get_barrier_semaphore()
        for nbr in (left, right):
            pl.semaphore_signal(
                barrier, inc=1,
                device_id=(nbr,), device_id_type=pl.DeviceIdType.MESH,
            )

        out_ref[...] = jnp.zeros((ROWS, D), jnp.float32)

        def issue_window(c, slot):
            base = c * CHUNK
            lo = my * V_PER

            def issue_one(j, cnt):
                g = idx_ref[j]
                owned = (g >> 13) == my

                @pl.when(owned)
                def _():
                    pltpu.make_async_copy(
                        table_ref.at[pl.ds(g - lo, 1), :],
                        out_ref.at[pl.ds(j, 1), :],
                        gather_sems.at[slot],
                    ).start()

                return cnt + owned.astype(jnp.int32)

            def gbody(i, cnt):
                j = base + i * 4
                for u in range(4):
                    cnt = issue_one(j + u, cnt)
                return cnt

            return lax.fori_loop(0, CHUNK // 4, gbody, jnp.int32(0))

        def wait_window(cnt, slot):
            wait_cp = pltpu.make_async_copy(
                table_ref.at[pl.ds(0, 1), :],
                out_ref.at[pl.ds(0, 1), :],
                gather_sems.at[slot],
            )

            def wbody(i, acc):
                wait_cp.wait()
                return acc

            lax.fori_loop(0, cnt, wbody, jnp.int32(0))

        cnt0 = issue_window(my, 0)

        pl.semaphore_wait(barrier, 2)

        wait_window(cnt0, 0)

        M = 2
        SEG = CHUNK // M
        rs_desc = {}

        def rs_rdma(dirn, s, h):
            if dirn == 0:
                c = lax.rem(my + (N_DEV - s), N_DEV)
                cols = pl.ds(0, HALF)
                land = land_r
                dev = right
            else:
                c = lax.rem(my + s, N_DEV)
                cols = pl.ds(HALF, HALF)
                land = land_l
                dev = left
            return pltpu.make_async_remote_copy(
                src_ref=out_ref.at[pl.ds(c * CHUNK + h * SEG, SEG), cols],
                dst_ref=land.at[s, pl.ds(h * SEG, SEG), :],
                send_sem=rs_send.at[dirn, s, h],
                recv_sem=rs_recv.at[dirn, s, h],
                device_id=(dev,),
                device_id_type=pl.DeviceIdType.MESH,
            )

        ag_desc = {}

        def ag_rdma(dirn, t, h):
            if dirn == 0:
                c = lax.rem(my + (N_DEV + 1 - t), N_DEV)
                cols = pl.ds(0, HALF)
                dev = right
            else:
                c = lax.rem(my + (N_DEV - 1 + t), N_DEV)
                cols = pl.ds(HALF, HALF)
                dev = left
            sl = pl.ds(c * CHUNK + h * SEG, SEG)
            return pltpu.make_async_remote_copy(
                src_ref=out_ref.at[sl, cols],
                dst_ref=out_ref.at[sl, cols],
                send_sem=ag_send.at[dirn, t, h],
                recv_sem=ag_recv.at[dirn, t, h],
                device_id=(dev,),
                device_id_type=pl.DeviceIdType.MESH,
            )

        for h in range(M):
            for dirn in (0, 1):
                rs_desc[(dirn, 0, h)] = rs_rdma(dirn, 0, h)
                rs_desc[(dirn, 0, h)].start()

        wcnt = {1: issue_window(lax.rem(my + (N_DEV - 1), N_DEV), 1)
                + issue_window(lax.rem(my + 1, N_DEV), 1)}

        for s in range(N_DEV - 1):
            crr = lax.rem(my + (N_DEV - s - 1), N_DEV)
            clr = lax.rem(my + s + 1, N_DEV)
            for h in range(M):
                for dirn in (0, 1):
                    rs_desc[(dirn, s, h)].wait_recv()
                    if dirn == 0 and h == 0 and s + 1 <= 4:
                        wait_window(wcnt[s + 1], s + 1)
                    if dirn == 0:
                        sl = pl.ds(crr * CHUNK + h * SEG, SEG)
                        out_ref[sl, pl.ds(0, HALF)] = (
                            out_ref[sl, pl.ds(0, HALF)]
                            + land_r[s, h * SEG:(h + 1) * SEG]
                        )
                    else:
                        sl = pl.ds(clr * CHUNK + h * SEG, SEG)
                        out_ref[sl, pl.ds(HALF, HALF)] = (
                            out_ref[sl, pl.ds(HALF, HALF)]
                            + land_l[s, h * SEG:(h + 1) * SEG]
                        )
                    if s < N_DEV - 2:
                        rs_desc[(dirn, s + 1, h)] = rs_rdma(dirn, s + 1, h)
                        rs_desc[(dirn, s + 1, h)].start()
                    else:
                        ag_desc[(dirn, 0, h)] = ag_rdma(dirn, 0, h)
                        ag_desc[(dirn, 0, h)].start()
            if s + 2 <= 4:
                cnt = issue_window(lax.rem(my + (N_DEV - s - 2), N_DEV), s + 2)
                if s + 2 < 4:
                    cnt = cnt + issue_window(lax.rem(my + s + 2, N_DEV), s + 2)
                wcnt[s + 2] = cnt

        for s in range(N_DEV - 1):
            for h in range(M):
                for dirn in (0, 1):
                    rs_desc[(dirn, s, h)].wait_send()

        for t in range(1, N_DEV - 1):
            for h in range(M):
                for dirn in (0, 1):
                    ag_desc[(dirn, t - 1, h)].wait_recv()
                    ag_desc[(dirn, t, h)] = ag_rdma(dirn, t, h)
                    ag_desc[(dirn, t, h)].start()
        for dirn in (0, 1):
            for h in range(M):
                ag_desc[(dirn, N_DEV - 2, h)].wait_recv()
        for t in range(N_DEV - 1):
            for h in range(M):
                for dirn in (0, 1):
                    ag_desc[(dirn, t, h)].wait_send()

    return pl.pallas_call(
        body,
        out_shape=jax.ShapeDtypeStruct((ROWS, D), jnp.float32),
        in_specs=[
            pl.BlockSpec(memory_space=pl.ANY),
            pl.BlockSpec(memory_space=pltpu.SMEM),
        ],
        out_specs=pl.BlockSpec(memory_space=pltpu.VMEM),
        scratch_shapes=[
            pltpu.VMEM((N_DEV - 1, CHUNK, HALF), jnp.float32),
            pltpu.VMEM((N_DEV - 1, CHUNK, HALF), jnp.float32),
            pltpu.SemaphoreType.DMA((5,)),
            pltpu.SemaphoreType.DMA((2, N_DEV - 1, 4)),
            pltpu.SemaphoreType.DMA((2, N_DEV - 1, 4)),
            pltpu.SemaphoreType.DMA((2, N_DEV - 1, 4)),
            pltpu.SemaphoreType.DMA((2, N_DEV - 1, 4)),
        ],
        compiler_params=pltpu.CompilerParams(collective_id=0),
    )(table, idx)
